# Initial kernel scaffold; baseline (speedup 1.0000x reference)
import jax
import jax.numpy as jnp
from jax.experimental import pallas as pl


def kernel(x, edge_index, batch, W_in, b_in, W_root, W_neigh, b_conv, ln_g, ln_b, W_out, b_out):
    raise NotImplementedError("write your pallas kernel here")


# speedup vs baseline: 3.2310x; 3.2310x over previous
import functools

import jax
import jax.numpy as jnp
from jax import lax
from jax.experimental import pallas as pl
from jax.experimental.pallas import tpu as pltpu
from jax.experimental.pallas import tpu_sc as plsc

_N = 10000
_E = 320000
_H = 128
_L = 3
_G = 256

_NPAD = 10240
_NC = 2
_NS = 16
_NW = _NC * _NS
_K = 128
_C = 80
_EPAD = _NW * _C * _K

_PK = 80
_PC = 4
_PROWS = 384
_PSTRIPE = _PROWS // _NS

_STRIPE = _NPAD // _NS


def _sc_mesh():
    return plsc.VectorSubcoreMesh(
        core_axis_name="c", subcore_axis_name="s", num_cores=_NC, num_subcores=_NS
    )


def _edge_agg_body(h_hbm, eidx_hbm, zeros_hbm, out_hbm, src_v, dst_v, rows_v, sem, agg_sp):
    c = lax.axis_index("c")
    s = lax.axis_index("s")
    w = c * _NS + s
    pltpu.sync_copy(zeros_hbm, rows_v)
    row0 = s * _STRIPE
    for t in range(_STRIPE // _K):
        pltpu.sync_copy(rows_v, agg_sp.at[pl.ds(row0 + t * _K, _K)])
    pltpu.sync_copy(eidx_hbm.at[0, w], src_v)
    pltpu.sync_copy(eidx_hbm.at[1, w], dst_v)
    plsc.subcore_barrier()

    def body(j, carry):
        pltpu.async_copy(h_hbm.at[src_v.at[j]], rows_v, sem).wait()
        pltpu.sync_copy(rows_v, agg_sp.at[dst_v.at[j]], add=True)
        return carry

    lax.fori_loop(0, _C, body, 0)
    plsc.subcore_barrier()
    for t in range(_STRIPE // _K):
        pltpu.sync_copy(agg_sp.at[pl.ds(row0 + t * _K, _K)], rows_v)
        pltpu.sync_copy(rows_v, out_hbm.at[c, pl.ds(row0 + t * _K, _K)])


def _edge_agg(h, eidx, zeros_k):
    return pl.kernel(
        _edge_agg_body,
        out_type=jax.ShapeDtypeStruct((_NC, _NPAD, _H), jnp.float32),
        mesh=_sc_mesh(),
        scratch_types=[
            pltpu.VMEM((_C, _K), jnp.int32),
            pltpu.VMEM((_C, _K), jnp.int32),
            pltpu.VMEM((_K, _H), jnp.float32),
            pltpu.SemaphoreType.DMA,
            pltpu.VMEM_SHARED((_NPAD, _H), jnp.float32),
        ],
    )(h, eidx, zeros_k)


def _pool_body(h_hbm, batch_hbm, ones_hbm, zeros_hbm, outp_hbm, outc_hbm,
               bidx_v, hrows_v, ones_v, zbuf_v, pool_sp, cnt_sp):
    c = lax.axis_index("c")
    s = lax.axis_index("s")
    w = c * _NS + s
    pltpu.sync_copy(batch_hbm.at[w], bidx_v)
    pltpu.sync_copy(ones_hbm, ones_v)
    pltpu.sync_copy(zeros_hbm, zbuf_v)
    pltpu.sync_copy(zbuf_v, pool_sp.at[pl.ds(s * _PSTRIPE, _PSTRIPE)])
    pltpu.sync_copy(zbuf_v, cnt_sp.at[pl.ds(s * _PSTRIPE, _PSTRIPE)])
    plsc.subcore_barrier()
    base = w * (_PC * _PK)
    for j in range(_PC):
        pltpu.sync_copy(h_hbm.at[pl.ds(base + j * _PK, _PK)], hrows_v)
        pltpu.sync_copy(hrows_v, pool_sp.at[bidx_v.at[j]], add=True)
        pltpu.sync_copy(ones_v, cnt_sp.at[bidx_v.at[j]], add=True)
    plsc.subcore_barrier()
    pltpu.sync_copy(pool_sp.at[pl.ds(s * _PSTRIPE, _PSTRIPE)], zbuf_v)
    pltpu.sync_copy(zbuf_v, outp_hbm.at[c, pl.ds(s * _PSTRIPE, _PSTRIPE)])
    pltpu.sync_copy(cnt_sp.at[pl.ds(s * _PSTRIPE, _PSTRIPE)], zbuf_v)
    pltpu.sync_copy(zbuf_v, outc_hbm.at[c, pl.ds(s * _PSTRIPE, _PSTRIPE)])


def _pool(h, bidx, ones_pk, zeros_ps):
    return pl.kernel(
        _pool_body,
        out_type=(
            jax.ShapeDtypeStruct((_NC, _PROWS, _H), jnp.float32),
            jax.ShapeDtypeStruct((_NC, _PROWS, _H), jnp.float32),
        ),
        mesh=_sc_mesh(),
        scratch_types=[
            pltpu.VMEM((_PC, _PK), jnp.int32),
            pltpu.VMEM((_PK, _H), jnp.float32),
            pltpu.VMEM((_PK, _H), jnp.float32),
            pltpu.VMEM((_PSTRIPE, _H), jnp.float32),
            pltpu.VMEM_SHARED((_PROWS, _H), jnp.float32),
            pltpu.VMEM_SHARED((_PROWS, _H), jnp.float32),
        ],
    )(h, bidx, ones_pk, zeros_ps)


_BLK = 1024


def _dense_in_body(x_ref, w_ref, b_ref, o_ref):
    o_ref[...] = (
        jnp.dot(x_ref[...], w_ref[...], preferred_element_type=jnp.float32)
        + b_ref[...]
    )


def _dense_in(x, w, b):
    return pl.pallas_call(
        _dense_in_body,
        grid=(_NPAD // _BLK,),
        in_specs=[
            pl.BlockSpec((_BLK, _H), lambda i: (i, 0)),
            pl.BlockSpec((_H, _H), lambda i: (0, 0)),
            pl.BlockSpec((1, _H), lambda i: (0, 0)),
        ],
        out_specs=pl.BlockSpec((_BLK, _H), lambda i: (i, 0)),
        out_shape=jax.ShapeDtypeStruct((_NPAD, _H), jnp.float32),
    )(x, w, b.reshape(1, _H))


def _conv_body(h_ref, a_ref, wr_ref, wn_ref, b_ref, g_ref, bb_ref, o_ref):
    h = h_ref[...]
    a = a_ref[0] + a_ref[1]
    y = (
        jnp.dot(h, wr_ref[...], preferred_element_type=jnp.float32)
        + jnp.dot(a, wn_ref[...], preferred_element_type=jnp.float32)
        + b_ref[...]
    )
    mu = jnp.mean(y, axis=-1, keepdims=True)
    d = y - mu
    var = jnp.mean(d * d, axis=-1, keepdims=True)
    y = d * lax.rsqrt(var + 1e-5) * g_ref[...] + bb_ref[...]
    o_ref[...] = jnp.maximum(y, 0.0)


def _conv(h, agg2, wr, wn, b, g, bb):
    return pl.pallas_call(
        _conv_body,
        grid=(_NPAD // _BLK,),
        in_specs=[
            pl.BlockSpec((_BLK, _H), lambda i: (i, 0)),
            pl.BlockSpec((_NC, _BLK, _H), lambda i: (0, i, 0)),
            pl.BlockSpec((_H, _H), lambda i: (0, 0)),
            pl.BlockSpec((_H, _H), lambda i: (0, 0)),
            pl.BlockSpec((1, _H), lambda i: (0, 0)),
            pl.BlockSpec((1, _H), lambda i: (0, 0)),
            pl.BlockSpec((1, _H), lambda i: (0, 0)),
        ],
        out_specs=pl.BlockSpec((_BLK, _H), lambda i: (i, 0)),
        out_shape=jax.ShapeDtypeStruct((_NPAD, _H), jnp.float32),
    )(h, agg2, wr, wn, b.reshape(1, _H), g.reshape(1, _H), bb.reshape(1, _H))


def _final_body(pp_ref, cc_ref, w_ref, b_ref, o_ref):
    p = pp_ref[0] + pp_ref[1]
    cnt = cc_ref[0] + cc_ref[1]
    pooled = p[:_G] / jnp.clip(cnt[:_G, 0:1], 1.0, None)
    o_ref[...] = (
        jnp.dot(pooled, w_ref[...], preferred_element_type=jnp.float32)
        + b_ref[...]
    )


def _final(pp, cc, w, b):
    return pl.pallas_call(
        _final_body,
        out_shape=jax.ShapeDtypeStruct((_G, _H), jnp.float32),
    )(pp, cc, w, b.reshape(1, _H))


def kernel(x, edge_index, batch, W_in, b_in, W_root, W_neigh, b_conv, ln_g, ln_b, W_out, b_out):
    f32 = jnp.float32
    x_pad = jnp.concatenate([x, jnp.zeros((_NPAD - _N, x.shape[1]), x.dtype)], axis=0)
    pad_e = jnp.full((2, _EPAD - _E), _N, jnp.int32)
    eidx = jnp.concatenate([edge_index.astype(jnp.int32), pad_e], axis=1)
    eidx = eidx.reshape(2, _NW, _C, _K)
    bidx = jnp.concatenate(
        [batch.astype(jnp.int32), jnp.full((_NPAD - _N,), _G, jnp.int32)]
    ).reshape(_NW, _PC, _PK)
    zeros_k = jnp.zeros((_K, _H), f32)
    zeros_ps = jnp.zeros((_PSTRIPE, _H), f32)
    ones_pk = jnp.ones((_PK, _H), f32)

    h = _dense_in(x_pad, W_in, b_in)
    for l in range(_L):
        agg2 = _edge_agg(h, eidx, zeros_k)
        h = _conv(h, agg2, W_root[l], W_neigh[l], b_conv[l], ln_g[l], ln_b[l])
    pp, cc = _pool(h, bidx, ones_pk, zeros_ps)
    return _final(pp, cc, W_out, b_out)


# speedup vs baseline: 6.3651x; 1.9700x over previous
import functools

import jax
import jax.numpy as jnp
from jax import lax
from jax.experimental import pallas as pl
from jax.experimental.pallas import tpu as pltpu
from jax.experimental.pallas import tpu_sc as plsc

_N = 10000
_E = 320000
_H = 128
_L = 3
_G = 256

_NPAD = 10240
_NC = 2
_NS = 16
_NW = _NC * _NS
_K = 64
_C = 160
_EPAD = _NW * _C * _K

_PK = 80
_PC = 4
_PROWS = 384
_PSTRIPE = _PROWS // _NS

_STRIPE = _NPAD // _NS


def _sc_mesh():
    return plsc.VectorSubcoreMesh(
        core_axis_name="c", subcore_axis_name="s", num_cores=_NC, num_subcores=_NS
    )


def _edge_agg_body(h_hbm, eidx_hbm, zeros_hbm, out_hbm, idx_v, rows0_v, rows1_v,
                   semi0, semi1, semg0, semg1, agg_sp):
    c = lax.axis_index("c")
    s = lax.axis_index("s")
    w = c * _NS + s
    pltpu.sync_copy(zeros_hbm, rows0_v)
    row0 = s * _STRIPE
    for t in range(_STRIPE // _K):
        pltpu.sync_copy(rows0_v, agg_sp.at[pl.ds(row0 + t * _K, _K)])
    plsc.subcore_barrier()

    idx0 = idx_v.at[0]
    idx1 = idx_v.at[1]

    pltpu.async_copy(eidx_hbm.at[w, 0], idx0, semi0)
    pltpu.async_copy(eidx_hbm.at[w, 1], idx1, semi1)
    pltpu.make_async_copy(eidx_hbm.at[w, 0], idx0, semi0).wait()
    pltpu.async_copy(h_hbm.at[idx0.at[0]], rows0_v, semg0)

    def step(j, io, ro, sio, sgo, ib, rb, sib, sgb):
        pltpu.make_async_copy(h_hbm.at[io.at[0]], ro, sgo).wait()
        pltpu.sync_copy(ro, agg_sp.at[io.at[1]], add=True)
        pltpu.async_copy(eidx_hbm.at[w, j + 1], io, sio)
        pltpu.make_async_copy(eidx_hbm.at[w, j], ib, sib).wait()
        pltpu.async_copy(h_hbm.at[ib.at[0]], rb, sgb)

    def body(jj, carry):
        j = 2 * jj + 1
        step(j, idx0, rows0_v, semi0, semg0, idx1, rows1_v, semi1, semg1)
        step(j + 1, idx1, rows1_v, semi1, semg1, idx0, rows0_v, semi0, semg0)
        return carry

    lax.fori_loop(0, _C // 2 - 1, body, 0)
    pltpu.make_async_copy(h_hbm.at[idx0.at[0]], rows0_v, semg0).wait()
    pltpu.sync_copy(rows0_v, agg_sp.at[idx0.at[1]], add=True)
    pltpu.make_async_copy(eidx_hbm.at[w, _C - 1], idx1, semi1).wait()
    pltpu.async_copy(h_hbm.at[idx1.at[0]], rows1_v, semg1)
    pltpu.make_async_copy(h_hbm.at[idx1.at[0]], rows1_v, semg1).wait()
    pltpu.sync_copy(rows1_v, agg_sp.at[idx1.at[1]], add=True)
    plsc.subcore_barrier()
    for t in range(_STRIPE // _K):
        pltpu.sync_copy(agg_sp.at[pl.ds(row0 + t * _K, _K)], rows0_v)
        pltpu.sync_copy(rows0_v, out_hbm.at[c, pl.ds(row0 + t * _K, _K)])


def _edge_agg(h, eidx, zeros_k):
    return pl.kernel(
        _edge_agg_body,
        out_type=jax.ShapeDtypeStruct((_NC, _NPAD, _H), jnp.float32),
        mesh=_sc_mesh(),
        scratch_types=[
            pltpu.VMEM((2, 2, _K), jnp.int32),
            pltpu.VMEM((_K, _H), jnp.float32),
            pltpu.VMEM((_K, _H), jnp.float32),
            pltpu.SemaphoreType.DMA,
            pltpu.SemaphoreType.DMA,
            pltpu.SemaphoreType.DMA,
            pltpu.SemaphoreType.DMA,
            pltpu.VMEM_SHARED((_NPAD, _H), jnp.float32),
        ],
    )(h, eidx, zeros_k)


def _pool_body(h_hbm, batch_hbm, ones_hbm, zeros_hbm, outp_hbm, outc_hbm,
               bidx_v, hrows_v, ones_v, zbuf_v, pool_sp, cnt_sp):
    c = lax.axis_index("c")
    s = lax.axis_index("s")
    w = c * _NS + s
    pltpu.sync_copy(batch_hbm.at[w], bidx_v)
    pltpu.sync_copy(ones_hbm, ones_v)
    pltpu.sync_copy(zeros_hbm, zbuf_v)
    pltpu.sync_copy(zbuf_v, pool_sp.at[pl.ds(s * _PSTRIPE, _PSTRIPE)])
    pltpu.sync_copy(zbuf_v, cnt_sp.at[pl.ds(s * _PSTRIPE, _PSTRIPE)])
    plsc.subcore_barrier()
    base = w * (_PC * _PK)
    for j in range(_PC):
        pltpu.sync_copy(h_hbm.at[pl.ds(base + j * _PK, _PK)], hrows_v)
        pltpu.sync_copy(hrows_v, pool_sp.at[bidx_v.at[j]], add=True)
        pltpu.sync_copy(ones_v, cnt_sp.at[bidx_v.at[j]], add=True)
    plsc.subcore_barrier()
    pltpu.sync_copy(pool_sp.at[pl.ds(s * _PSTRIPE, _PSTRIPE)], zbuf_v)
    pltpu.sync_copy(zbuf_v, outp_hbm.at[c, pl.ds(s * _PSTRIPE, _PSTRIPE)])
    pltpu.sync_copy(cnt_sp.at[pl.ds(s * _PSTRIPE, _PSTRIPE)], zbuf_v)
    pltpu.sync_copy(zbuf_v, outc_hbm.at[c, pl.ds(s * _PSTRIPE, _PSTRIPE)])


def _pool(h, bidx, ones_pk, zeros_ps):
    return pl.kernel(
        _pool_body,
        out_type=(
            jax.ShapeDtypeStruct((_NC, _PROWS, _H), jnp.float32),
            jax.ShapeDtypeStruct((_NC, _PROWS, _H), jnp.float32),
        ),
        mesh=_sc_mesh(),
        scratch_types=[
            pltpu.VMEM((_PC, _PK), jnp.int32),
            pltpu.VMEM((_PK, _H), jnp.float32),
            pltpu.VMEM((_PK, _H), jnp.float32),
            pltpu.VMEM((_PSTRIPE, _H), jnp.float32),
            pltpu.VMEM_SHARED((_PROWS, _H), jnp.float32),
            pltpu.VMEM_SHARED((_PROWS, _H), jnp.float32),
        ],
    )(h, bidx, ones_pk, zeros_ps)


_BLK = 1024


def _dense_in_body(x_ref, w_ref, b_ref, o_ref):
    o_ref[...] = (
        jnp.dot(x_ref[...], w_ref[...], preferred_element_type=jnp.float32)
        + b_ref[...]
    )


def _dense_in(x, w, b):
    return pl.pallas_call(
        _dense_in_body,
        grid=(_NPAD // _BLK,),
        in_specs=[
            pl.BlockSpec((_BLK, _H), lambda i: (i, 0)),
            pl.BlockSpec((_H, _H), lambda i: (0, 0)),
            pl.BlockSpec((1, _H), lambda i: (0, 0)),
        ],
        out_specs=pl.BlockSpec((_BLK, _H), lambda i: (i, 0)),
        out_shape=jax.ShapeDtypeStruct((_NPAD, _H), jnp.float32),
    )(x, w, b.reshape(1, _H))


def _conv_body(h_ref, a_ref, wr_ref, wn_ref, b_ref, g_ref, bb_ref, o_ref):
    h = h_ref[...]
    a = a_ref[0] + a_ref[1]
    y = (
        jnp.dot(h, wr_ref[...], preferred_element_type=jnp.float32)
        + jnp.dot(a, wn_ref[...], preferred_element_type=jnp.float32)
        + b_ref[...]
    )
    mu = jnp.mean(y, axis=-1, keepdims=True)
    d = y - mu
    var = jnp.mean(d * d, axis=-1, keepdims=True)
    y = d * lax.rsqrt(var + 1e-5) * g_ref[...] + bb_ref[...]
    o_ref[...] = jnp.maximum(y, 0.0)


def _conv(h, agg2, wr, wn, b, g, bb):
    return pl.pallas_call(
        _conv_body,
        grid=(_NPAD // _BLK,),
        in_specs=[
            pl.BlockSpec((_BLK, _H), lambda i: (i, 0)),
            pl.BlockSpec((_NC, _BLK, _H), lambda i: (0, i, 0)),
            pl.BlockSpec((_H, _H), lambda i: (0, 0)),
            pl.BlockSpec((_H, _H), lambda i: (0, 0)),
            pl.BlockSpec((1, _H), lambda i: (0, 0)),
            pl.BlockSpec((1, _H), lambda i: (0, 0)),
            pl.BlockSpec((1, _H), lambda i: (0, 0)),
        ],
        out_specs=pl.BlockSpec((_BLK, _H), lambda i: (i, 0)),
        out_shape=jax.ShapeDtypeStruct((_NPAD, _H), jnp.float32),
    )(h, agg2, wr, wn, b.reshape(1, _H), g.reshape(1, _H), bb.reshape(1, _H))


def _final_body(pp_ref, cc_ref, w_ref, b_ref, o_ref):
    p = pp_ref[0] + pp_ref[1]
    cnt = cc_ref[0] + cc_ref[1]
    pooled = p[:_G] / jnp.clip(cnt[:_G, 0:1], 1.0, None)
    o_ref[...] = (
        jnp.dot(pooled, w_ref[...], preferred_element_type=jnp.float32)
        + b_ref[...]
    )


def _final(pp, cc, w, b):
    return pl.pallas_call(
        _final_body,
        out_shape=jax.ShapeDtypeStruct((_G, _H), jnp.float32),
    )(pp, cc, w, b.reshape(1, _H))


def kernel(x, edge_index, batch, W_in, b_in, W_root, W_neigh, b_conv, ln_g, ln_b, W_out, b_out):
    f32 = jnp.float32
    x_pad = jnp.concatenate([x, jnp.zeros((_NPAD - _N, x.shape[1]), x.dtype)], axis=0)
    pad_ids = _N + (jnp.arange(_EPAD - _E, dtype=jnp.int32) % (_NPAD - _N))
    src_pad = jnp.concatenate([edge_index[0].astype(jnp.int32), pad_ids])
    dst_pad = jnp.concatenate([edge_index[1].astype(jnp.int32), pad_ids])
    eidx = jnp.stack(
        [src_pad.reshape(_NW, _C, _K), dst_pad.reshape(_NW, _C, _K)], axis=2
    )
    bidx = jnp.concatenate(
        [batch.astype(jnp.int32), jnp.full((_NPAD - _N,), _G, jnp.int32)]
    ).reshape(_NW, _PC, _PK)
    zeros_k = jnp.zeros((_K, _H), f32)
    zeros_ps = jnp.zeros((_PSTRIPE, _H), f32)
    ones_pk = jnp.ones((_PK, _H), f32)

    h = _dense_in(x_pad, W_in, b_in)
    for l in range(_L):
        agg2 = _edge_agg(h, eidx, zeros_k)
        h = _conv(h, agg2, W_root[l], W_neigh[l], b_conv[l], ln_g[l], ln_b[l])
    pp, cc = _pool(h, bidx, ones_pk, zeros_ps)
    return _final(pp, cc, W_out, b_out)


# speedup vs baseline: 7.7618x; 1.2194x over previous
import functools

import jax
import jax.numpy as jnp
from jax import lax
from jax.experimental import pallas as pl
from jax.experimental.pallas import tpu as pltpu
from jax.experimental.pallas import tpu_sc as plsc

_N = 10000
_E = 320000
_H = 128
_L = 3
_G = 256

_NPAD = 10240
_NC = 2
_NS = 16
_NW = _NC * _NS
_K = 128
_C = 80
_EPAD = _NW * _C * _K

_PK = 80
_PC = 4
_PROWS = 384
_PSTRIPE = _PROWS // _NS

_STRIPE = _NPAD // _NS


def _sc_mesh():
    return plsc.VectorSubcoreMesh(
        core_axis_name="c", subcore_axis_name="s", num_cores=_NC, num_subcores=_NS
    )


def _edge_agg_body(h_hbm, eidx_hbm, zeros_hbm, out_hbm, idx_v, rows0_v, rows1_v,
                   semi0, semi1, semg0, semg1, agg_sp):
    c = lax.axis_index("c")
    s = lax.axis_index("s")
    w = c * _NS + s
    row0 = s * _STRIPE
    for t in range(_STRIPE // _K):
        pltpu.sync_copy(zeros_hbm, agg_sp.at[pl.ds(row0 + t * _K, _K)])
    plsc.subcore_barrier()

    idx0 = idx_v.at[0]
    idx1 = idx_v.at[1]

    pltpu.async_copy(eidx_hbm.at[w, 0], idx0, semi0)
    pltpu.async_copy(eidx_hbm.at[w, 1], idx1, semi1)
    pltpu.make_async_copy(eidx_hbm.at[w, 0], idx0, semi0).wait()
    pltpu.async_copy(h_hbm.at[idx0.at[0]], rows0_v, semg0)

    def step(j, io, ro, sio, sgo, ib, rb, sib, sgb):
        pltpu.make_async_copy(h_hbm.at[io.at[0]], ro, sgo).wait()
        pltpu.sync_copy(ro, agg_sp.at[io.at[1]], add=True)
        pltpu.async_copy(eidx_hbm.at[w, j + 1], io, sio)
        pltpu.make_async_copy(eidx_hbm.at[w, j], ib, sib).wait()
        pltpu.async_copy(h_hbm.at[ib.at[0]], rb, sgb)

    def body(jj, carry):
        j = 2 * jj + 1
        step(j, idx0, rows0_v, semi0, semg0, idx1, rows1_v, semi1, semg1)
        step(j + 1, idx1, rows1_v, semi1, semg1, idx0, rows0_v, semi0, semg0)
        return carry

    lax.fori_loop(0, _C // 2 - 1, body, 0)
    pltpu.make_async_copy(h_hbm.at[idx0.at[0]], rows0_v, semg0).wait()
    pltpu.sync_copy(rows0_v, agg_sp.at[idx0.at[1]], add=True)
    pltpu.make_async_copy(eidx_hbm.at[w, _C - 1], idx1, semi1).wait()
    pltpu.async_copy(h_hbm.at[idx1.at[0]], rows1_v, semg1)
    pltpu.make_async_copy(h_hbm.at[idx1.at[0]], rows1_v, semg1).wait()
    pltpu.sync_copy(rows1_v, agg_sp.at[idx1.at[1]], add=True)
    plsc.subcore_barrier()
    pltpu.sync_copy(agg_sp.at[pl.ds(row0, _STRIPE)], out_hbm.at[c, pl.ds(row0, _STRIPE)])


def _edge_agg(h, eidx, zeros_k):
    return pl.kernel(
        _edge_agg_body,
        out_type=jax.ShapeDtypeStruct((_NC, _NPAD, _H), jnp.float32),
        mesh=_sc_mesh(),
        scratch_types=[
            pltpu.VMEM((2, 2, _K), jnp.int32),
            pltpu.VMEM((_K, _H), jnp.float32),
            pltpu.VMEM((_K, _H), jnp.float32),
            pltpu.SemaphoreType.DMA,
            pltpu.SemaphoreType.DMA,
            pltpu.SemaphoreType.DMA,
            pltpu.SemaphoreType.DMA,
            pltpu.VMEM_SHARED((_NPAD, _H), jnp.float32),
        ],
    )(h, eidx, zeros_k)


def _pool_body(h_hbm, batch_hbm, ones_hbm, zeros_hbm, outp_hbm, outc_hbm,
               bidx_v, hrows_v, ones_v, zbuf_v, pool_sp, cnt_sp):
    c = lax.axis_index("c")
    s = lax.axis_index("s")
    w = c * _NS + s
    pltpu.sync_copy(batch_hbm.at[w], bidx_v)
    pltpu.sync_copy(ones_hbm, ones_v)
    pltpu.sync_copy(zeros_hbm, zbuf_v)
    pltpu.sync_copy(zbuf_v, pool_sp.at[pl.ds(s * _PSTRIPE, _PSTRIPE)])
    pltpu.sync_copy(zbuf_v, cnt_sp.at[pl.ds(s * _PSTRIPE, _PSTRIPE)])
    plsc.subcore_barrier()
    base = w * (_PC * _PK)
    for j in range(_PC):
        pltpu.sync_copy(h_hbm.at[pl.ds(base + j * _PK, _PK)], hrows_v)
        pltpu.sync_copy(hrows_v, pool_sp.at[bidx_v.at[j]], add=True)
        pltpu.sync_copy(ones_v, cnt_sp.at[bidx_v.at[j]], add=True)
    plsc.subcore_barrier()
    pltpu.sync_copy(pool_sp.at[pl.ds(s * _PSTRIPE, _PSTRIPE)], zbuf_v)
    pltpu.sync_copy(zbuf_v, outp_hbm.at[c, pl.ds(s * _PSTRIPE, _PSTRIPE)])
    pltpu.sync_copy(cnt_sp.at[pl.ds(s * _PSTRIPE, _PSTRIPE)], zbuf_v)
    pltpu.sync_copy(zbuf_v, outc_hbm.at[c, pl.ds(s * _PSTRIPE, _PSTRIPE)])


def _pool(h, bidx, ones_pk, zeros_ps):
    return pl.kernel(
        _pool_body,
        out_type=(
            jax.ShapeDtypeStruct((_NC, _PROWS, _H), jnp.float32),
            jax.ShapeDtypeStruct((_NC, _PROWS, _H), jnp.float32),
        ),
        mesh=_sc_mesh(),
        scratch_types=[
            pltpu.VMEM((_PC, _PK), jnp.int32),
            pltpu.VMEM((_PK, _H), jnp.float32),
            pltpu.VMEM((_PK, _H), jnp.float32),
            pltpu.VMEM((_PSTRIPE, _H), jnp.float32),
            pltpu.VMEM_SHARED((_PROWS, _H), jnp.float32),
            pltpu.VMEM_SHARED((_PROWS, _H), jnp.float32),
        ],
    )(h, bidx, ones_pk, zeros_ps)


_BLK = 1024


def _dense_in_body(x_ref, w_ref, b_ref, o_ref):
    o_ref[...] = (
        jnp.dot(x_ref[...], w_ref[...], preferred_element_type=jnp.float32)
        + b_ref[...]
    )


def _dense_in(x, w, b):
    return pl.pallas_call(
        _dense_in_body,
        grid=(_NPAD // _BLK,),
        in_specs=[
            pl.BlockSpec((_BLK, _H), lambda i: (i, 0)),
            pl.BlockSpec((_H, _H), lambda i: (0, 0)),
            pl.BlockSpec((1, _H), lambda i: (0, 0)),
        ],
        out_specs=pl.BlockSpec((_BLK, _H), lambda i: (i, 0)),
        out_shape=jax.ShapeDtypeStruct((_NPAD, _H), jnp.float32),
    )(x, w, b.reshape(1, _H))


def _conv_body(h_ref, a_ref, wr_ref, wn_ref, b_ref, g_ref, bb_ref, o_ref):
    h = h_ref[...]
    a = a_ref[0] + a_ref[1]
    y = (
        jnp.dot(h, wr_ref[...], preferred_element_type=jnp.float32)
        + jnp.dot(a, wn_ref[...], preferred_element_type=jnp.float32)
        + b_ref[...]
    )
    mu = jnp.mean(y, axis=-1, keepdims=True)
    d = y - mu
    var = jnp.mean(d * d, axis=-1, keepdims=True)
    y = d * lax.rsqrt(var + 1e-5) * g_ref[...] + bb_ref[...]
    o_ref[...] = jnp.maximum(y, 0.0)


def _conv(h, agg2, wr, wn, b, g, bb):
    return pl.pallas_call(
        _conv_body,
        grid=(_NPAD // _BLK,),
        in_specs=[
            pl.BlockSpec((_BLK, _H), lambda i: (i, 0)),
            pl.BlockSpec((_NC, _BLK, _H), lambda i: (0, i, 0)),
            pl.BlockSpec((_H, _H), lambda i: (0, 0)),
            pl.BlockSpec((_H, _H), lambda i: (0, 0)),
            pl.BlockSpec((1, _H), lambda i: (0, 0)),
            pl.BlockSpec((1, _H), lambda i: (0, 0)),
            pl.BlockSpec((1, _H), lambda i: (0, 0)),
        ],
        out_specs=pl.BlockSpec((_BLK, _H), lambda i: (i, 0)),
        out_shape=jax.ShapeDtypeStruct((_NPAD, _H), jnp.float32),
    )(h, agg2, wr, wn, b.reshape(1, _H), g.reshape(1, _H), bb.reshape(1, _H))


def _final_body(pp_ref, cc_ref, w_ref, b_ref, o_ref):
    p = pp_ref[0] + pp_ref[1]
    cnt = cc_ref[0] + cc_ref[1]
    pooled = p[:_G] / jnp.clip(cnt[:_G, 0:1], 1.0, None)
    o_ref[...] = (
        jnp.dot(pooled, w_ref[...], preferred_element_type=jnp.float32)
        + b_ref[...]
    )


def _final(pp, cc, w, b):
    return pl.pallas_call(
        _final_body,
        out_shape=jax.ShapeDtypeStruct((_G, _H), jnp.float32),
    )(pp, cc, w, b.reshape(1, _H))


def kernel(x, edge_index, batch, W_in, b_in, W_root, W_neigh, b_conv, ln_g, ln_b, W_out, b_out):
    f32 = jnp.float32
    x_pad = jnp.concatenate([x, jnp.zeros((_NPAD - _N, x.shape[1]), x.dtype)], axis=0)
    pad_ids = _N + (jnp.arange(_EPAD - _E, dtype=jnp.int32) % (_NPAD - _N))
    src_pad = jnp.concatenate([edge_index[0].astype(jnp.int32), pad_ids])
    dst_pad = jnp.concatenate([edge_index[1].astype(jnp.int32), pad_ids])
    eidx = jnp.stack(
        [src_pad.reshape(_NW, _C, _K), dst_pad.reshape(_NW, _C, _K)], axis=2
    )
    bidx = jnp.concatenate(
        [batch.astype(jnp.int32), jnp.full((_NPAD - _N,), _G, jnp.int32)]
    ).reshape(_NW, _PC, _PK)
    zeros_k = jnp.zeros((_K, _H), f32)
    zeros_ps = jnp.zeros((_PSTRIPE, _H), f32)
    ones_pk = jnp.ones((_PK, _H), f32)

    h = _dense_in(x_pad, W_in, b_in)
    for l in range(_L):
        agg2 = _edge_agg(h, eidx, zeros_k)
        h = _conv(h, agg2, W_root[l], W_neigh[l], b_conv[l], ln_g[l], ln_b[l])
    pp, cc = _pool(h, bidx, ones_pk, zeros_ps)
    return _final(pp, cc, W_out, b_out)


# speedup vs baseline: 8.2356x; 1.0610x over previous
import functools

import jax
import jax.numpy as jnp
from jax import lax
from jax.experimental import pallas as pl
from jax.experimental.pallas import tpu as pltpu
from jax.experimental.pallas import tpu_sc as plsc

_N = 10000
_E = 320000
_H = 128
_L = 3
_G = 256

_NPAD = 10240
_NC = 2
_NS = 16
_NW = _NC * _NS
_K = 80
_C = 128
_RB = 4
_IB = 8
_EPAD = _NW * _C * _K

_PK = 80
_PC = 4
_PROWS = 384
_PSTRIPE = _PROWS // _NS

_STRIPE = _NPAD // _NS


def _sc_mesh():
    return plsc.VectorSubcoreMesh(
        core_axis_name="c", subcore_axis_name="s", num_cores=_NC, num_subcores=_NS
    )


def _edge_agg_body(h_hbm, eidx_hbm, zeros_hbm, out_hbm, idx_v, rows_v,
                   semi, semg, sems, agg_sp):
    c = lax.axis_index("c")
    s = lax.axis_index("s")
    w = c * _NS + s
    row0 = s * _STRIPE
    nz = _STRIPE // _K
    for t in range(nz):
        pltpu.async_copy(zeros_hbm, agg_sp.at[pl.ds(row0 + t * _K, _K)], semi.at[t % _IB])
    for t in range(nz):
        pltpu.make_async_copy(zeros_hbm, agg_sp.at[pl.ds(row0 + t * _K, _K)], semi.at[t % _IB]).wait()
    plsc.subcore_barrier()

    def idx_load(j, slot):
        pltpu.async_copy(eidx_hbm.at[w, j], idx_v.at[slot], semi.at[slot])

    def idx_wait(j, slot):
        pltpu.make_async_copy(eidx_hbm.at[w, j], idx_v.at[slot], semi.at[slot]).wait()

    def gather(j, u8, r):
        pltpu.async_copy(h_hbm.at[idx_v.at[u8, 0]], rows_v.at[r], semg.at[r])

    def gather_wait(u8, r):
        pltpu.make_async_copy(h_hbm.at[idx_v.at[u8, 0]], rows_v.at[r], semg.at[r]).wait()

    def scatter(u8, r):
        pltpu.async_copy(rows_v.at[r], agg_sp.at[idx_v.at[u8, 1]], sems.at[r], add=True)

    def scatter_wait(u8, r):
        pltpu.make_async_copy(rows_v.at[r], agg_sp.at[idx_v.at[u8, 1]], sems.at[r]).wait()

    def iteration(j, u, skip_prev=False, skip_s4=False, do_idx=True):
        up = (u - 1) % _IB
        if not skip_prev:
            gather_wait(up, up % _RB)
            scatter(up, up % _RB)
        idx_wait(j, u)
        if not skip_s4:
            scatter_wait(u, u % _RB)
        gather(j, u, u % _RB)
        if do_idx:
            idx_load(j + _RB, (u + _RB) % _IB)

    for t in range(_RB):
        idx_load(t, t)
    iteration(0, 0, skip_prev=True, skip_s4=True)
    for t in range(1, _RB):
        iteration(t, t, skip_s4=True)
    for t in range(_RB, _IB):
        iteration(t, t)

    def body(bb, carry):
        j0 = _IB + _IB * bb
        for u in range(_IB):
            iteration(j0 + u, u)
        return carry

    lax.fori_loop(0, (_C - 2 * _IB) // _IB, body, 0)
    for t in range(_IB):
        j = _C - _IB + t
        iteration(j, t, do_idx=(t < _RB))
    last_u = (_C - 1) % _IB
    gather_wait(last_u, last_u % _RB)
    scatter(last_u, last_u % _RB)
    for r in range(_RB):
        scatter_wait(0, r)
    plsc.subcore_barrier()
    pltpu.sync_copy(agg_sp.at[pl.ds(row0, _STRIPE)], out_hbm.at[c, pl.ds(row0, _STRIPE)])


def _edge_agg(h, eidx, zeros_k):
    return pl.kernel(
        _edge_agg_body,
        out_type=jax.ShapeDtypeStruct((_NC, _NPAD, _H), jnp.float32),
        mesh=_sc_mesh(),
        scratch_types=[
            pltpu.VMEM((_IB, 2, _K), jnp.int32),
            pltpu.VMEM((_RB, _K, _H), jnp.float32),
            pltpu.SemaphoreType.DMA((_IB,)),
            pltpu.SemaphoreType.DMA((_RB,)),
            pltpu.SemaphoreType.DMA((_RB,)),
            pltpu.VMEM_SHARED((_NPAD, _H), jnp.float32),
        ],
    )(h, eidx, zeros_k)


def _pool_body(h_hbm, batch_hbm, ones_hbm, zeros_hbm, outp_hbm, outc_hbm,
               bidx_v, hrows_v, ones_v, zbuf_v, pool_sp, cnt_sp):
    c = lax.axis_index("c")
    s = lax.axis_index("s")
    w = c * _NS + s
    pltpu.sync_copy(batch_hbm.at[w], bidx_v)
    pltpu.sync_copy(ones_hbm, ones_v)
    pltpu.sync_copy(zeros_hbm, zbuf_v)
    pltpu.sync_copy(zbuf_v, pool_sp.at[pl.ds(s * _PSTRIPE, _PSTRIPE)])
    pltpu.sync_copy(zbuf_v, cnt_sp.at[pl.ds(s * _PSTRIPE, _PSTRIPE)])
    plsc.subcore_barrier()
    base = w * (_PC * _PK)
    for j in range(_PC):
        pltpu.sync_copy(h_hbm.at[pl.ds(base + j * _PK, _PK)], hrows_v)
        pltpu.sync_copy(hrows_v, pool_sp.at[bidx_v.at[j]], add=True)
        pltpu.sync_copy(ones_v, cnt_sp.at[bidx_v.at[j]], add=True)
    plsc.subcore_barrier()
    pltpu.sync_copy(pool_sp.at[pl.ds(s * _PSTRIPE, _PSTRIPE)], zbuf_v)
    pltpu.sync_copy(zbuf_v, outp_hbm.at[c, pl.ds(s * _PSTRIPE, _PSTRIPE)])
    pltpu.sync_copy(cnt_sp.at[pl.ds(s * _PSTRIPE, _PSTRIPE)], zbuf_v)
    pltpu.sync_copy(zbuf_v, outc_hbm.at[c, pl.ds(s * _PSTRIPE, _PSTRIPE)])


def _pool(h, bidx, ones_pk, zeros_ps):
    return pl.kernel(
        _pool_body,
        out_type=(
            jax.ShapeDtypeStruct((_NC, _PROWS, _H), jnp.float32),
            jax.ShapeDtypeStruct((_NC, _PROWS, _H), jnp.float32),
        ),
        mesh=_sc_mesh(),
        scratch_types=[
            pltpu.VMEM((_PC, _PK), jnp.int32),
            pltpu.VMEM((_PK, _H), jnp.float32),
            pltpu.VMEM((_PK, _H), jnp.float32),
            pltpu.VMEM((_PSTRIPE, _H), jnp.float32),
            pltpu.VMEM_SHARED((_PROWS, _H), jnp.float32),
            pltpu.VMEM_SHARED((_PROWS, _H), jnp.float32),
        ],
    )(h, bidx, ones_pk, zeros_ps)


_BLK = 1024


def _dense_in_body(x_ref, w_ref, b_ref, o_ref):
    o_ref[...] = (
        jnp.dot(x_ref[...], w_ref[...], preferred_element_type=jnp.float32)
        + b_ref[...]
    )


def _dense_in(x, w, b):
    return pl.pallas_call(
        _dense_in_body,
        grid=(_NPAD // _BLK,),
        in_specs=[
            pl.BlockSpec((_BLK, _H), lambda i: (i, 0)),
            pl.BlockSpec((_H, _H), lambda i: (0, 0)),
            pl.BlockSpec((1, _H), lambda i: (0, 0)),
        ],
        out_specs=pl.BlockSpec((_BLK, _H), lambda i: (i, 0)),
        out_shape=jax.ShapeDtypeStruct((_NPAD, _H), jnp.float32),
    )(x, w, b.reshape(1, _H))


def _conv_body(h_ref, a_ref, wr_ref, wn_ref, b_ref, g_ref, bb_ref, o_ref):
    h = h_ref[...]
    a = a_ref[0] + a_ref[1]
    y = (
        jnp.dot(h, wr_ref[...], preferred_element_type=jnp.float32)
        + jnp.dot(a, wn_ref[...], preferred_element_type=jnp.float32)
        + b_ref[...]
    )
    mu = jnp.mean(y, axis=-1, keepdims=True)
    d = y - mu
    var = jnp.mean(d * d, axis=-1, keepdims=True)
    y = d * lax.rsqrt(var + 1e-5) * g_ref[...] + bb_ref[...]
    o_ref[...] = jnp.maximum(y, 0.0)


def _conv(h, agg2, wr, wn, b, g, bb):
    return pl.pallas_call(
        _conv_body,
        grid=(_NPAD // _BLK,),
        in_specs=[
            pl.BlockSpec((_BLK, _H), lambda i: (i, 0)),
            pl.BlockSpec((_NC, _BLK, _H), lambda i: (0, i, 0)),
            pl.BlockSpec((_H, _H), lambda i: (0, 0)),
            pl.BlockSpec((_H, _H), lambda i: (0, 0)),
            pl.BlockSpec((1, _H), lambda i: (0, 0)),
            pl.BlockSpec((1, _H), lambda i: (0, 0)),
            pl.BlockSpec((1, _H), lambda i: (0, 0)),
        ],
        out_specs=pl.BlockSpec((_BLK, _H), lambda i: (i, 0)),
        out_shape=jax.ShapeDtypeStruct((_NPAD, _H), jnp.float32),
    )(h, agg2, wr, wn, b.reshape(1, _H), g.reshape(1, _H), bb.reshape(1, _H))


def _final_body(pp_ref, cc_ref, w_ref, b_ref, o_ref):
    p = pp_ref[0] + pp_ref[1]
    cnt = cc_ref[0] + cc_ref[1]
    pooled = p[:_G] / jnp.clip(cnt[:_G, 0:1], 1.0, None)
    o_ref[...] = (
        jnp.dot(pooled, w_ref[...], preferred_element_type=jnp.float32)
        + b_ref[...]
    )


def _final(pp, cc, w, b):
    return pl.pallas_call(
        _final_body,
        out_shape=jax.ShapeDtypeStruct((_G, _H), jnp.float32),
    )(pp, cc, w, b.reshape(1, _H))


def kernel(x, edge_index, batch, W_in, b_in, W_root, W_neigh, b_conv, ln_g, ln_b, W_out, b_out):
    f32 = jnp.float32
    x_pad = jnp.concatenate([x, jnp.zeros((_NPAD - _N, x.shape[1]), x.dtype)], axis=0)
    pad_ids = _N + (jnp.arange(_EPAD - _E, dtype=jnp.int32) % (_NPAD - _N))
    src_pad = jnp.concatenate([edge_index[0].astype(jnp.int32), pad_ids])
    dst_pad = jnp.concatenate([edge_index[1].astype(jnp.int32), pad_ids])
    eidx = jnp.stack(
        [src_pad.reshape(_NW, _C, _K), dst_pad.reshape(_NW, _C, _K)], axis=2
    )
    bidx = jnp.concatenate(
        [batch.astype(jnp.int32), jnp.full((_NPAD - _N,), _G, jnp.int32)]
    ).reshape(_NW, _PC, _PK)
    zeros_k = jnp.zeros((_K, _H), f32)
    zeros_ps = jnp.zeros((_PSTRIPE, _H), f32)
    ones_pk = jnp.ones((_PK, _H), f32)

    h = _dense_in(x_pad, W_in, b_in)
    for l in range(_L):
        agg2 = _edge_agg(h, eidx, zeros_k)
        h = _conv(h, agg2, W_root[l], W_neigh[l], b_conv[l], ln_g[l], ln_b[l])
    pp, cc = _pool(h, bidx, ones_pk, zeros_ps)
    return _final(pp, cc, W_out, b_out)


# speedup vs baseline: 11.2481x; 1.3658x over previous
import functools

import jax
import jax.numpy as jnp
from jax import lax
from jax.experimental import pallas as pl
from jax.experimental.pallas import tpu as pltpu
from jax.experimental.pallas import tpu_sc as plsc

_N = 10000
_E = 320000
_H = 128
_L = 3
_G = 256

_NPAD = 10240
_NC = 2
_NS = 16
_NW = _NC * _NS
_K = 64
_C = 160
_RB = 5
_GD = 3
_IB = 10
_EPAD = _NW * _C * _K

_PK = 80
_PC = 4
_PROWS = 384
_PSTRIPE = _PROWS // _NS

_STRIPE = _NPAD // _NS


def _sc_mesh():
    return plsc.VectorSubcoreMesh(
        core_axis_name="c", subcore_axis_name="s", num_cores=_NC, num_subcores=_NS
    )


def _edge_agg_body(h_hbm, eidx_hbm, zeros_hbm, out_hbm, idx_v, rows_v,
                   semi, semg, sems, agg_sp):
    c = lax.axis_index("c")
    s = lax.axis_index("s")
    w = c * _NS + s
    row0 = s * _STRIPE
    nz = _STRIPE // _K
    for t in range(nz):
        pltpu.async_copy(zeros_hbm, agg_sp.at[pl.ds(row0 + t * _K, _K)], semi.at[t % _IB])
    for t in range(nz):
        pltpu.make_async_copy(zeros_hbm, agg_sp.at[pl.ds(row0 + t * _K, _K)], semi.at[t % _IB]).wait()
    plsc.subcore_barrier()

    def idx_load(j, slot):
        pltpu.async_copy(eidx_hbm.at[w, j], idx_v.at[slot], semi.at[slot])

    def idx_wait(j, slot):
        pltpu.make_async_copy(eidx_hbm.at[w, j], idx_v.at[slot], semi.at[slot]).wait()

    def gather(j, u8, r):
        pltpu.async_copy(h_hbm.at[idx_v.at[u8, 0]], rows_v.at[r], semg.at[r])

    def gather_wait(u8, r):
        pltpu.make_async_copy(h_hbm.at[idx_v.at[u8, 0]], rows_v.at[r], semg.at[r]).wait()

    def scatter(u8, r):
        pltpu.async_copy(rows_v.at[r], agg_sp.at[idx_v.at[u8, 1]], sems.at[r], add=True)

    def scatter_wait(r):
        pltpu.make_async_copy(rows_v.at[r], agg_sp.at[idx_v.at[0, 1]], sems.at[r]).wait()

    def iteration(j, u, skip_b=False, skip_ef=False, do_idx=True):
        idx_wait(j, u)
        if not skip_b:
            scatter_wait(u % _RB)
        gather(j, u, u % _RB)
        if do_idx:
            idx_load(j + _RB, (u + _RB) % _IB)
        if not skip_ef:
            ug = (u - _GD) % _IB
            gather_wait(ug, ug % _RB)
            scatter(ug, ug % _RB)

    for t in range(_RB):
        idx_load(t, t)
    for t in range(_IB):
        iteration(t, t, skip_b=(t < _RB), skip_ef=(t < _GD))

    def body(bb, carry):
        j0 = _IB + _IB * bb
        for u in range(_IB):
            iteration(j0 + u, u)
        return carry

    lax.fori_loop(0, (_C - 2 * _IB) // _IB, body, 0)
    for t in range(_IB):
        j = _C - _IB + t
        iteration(j, t, do_idx=(t < _IB - _RB))
    for t in range(_GD):
        j = _C - _GD + t
        u = j % _IB
        gather_wait(u, u % _RB)
        scatter(u, u % _RB)
    for r in range(_RB):
        scatter_wait(r)
    plsc.subcore_barrier()
    pltpu.sync_copy(agg_sp.at[pl.ds(row0, _STRIPE)], out_hbm.at[c, pl.ds(row0, _STRIPE)])


def _edge_agg(h, eidx, zeros_k):
    return pl.kernel(
        _edge_agg_body,
        out_type=jax.ShapeDtypeStruct((_NC, _NPAD, _H), jnp.float32),
        mesh=_sc_mesh(),
        scratch_types=[
            pltpu.VMEM((_IB, 2, _K), jnp.int32),
            pltpu.VMEM((_RB, _K, _H), jnp.float32),
            pltpu.SemaphoreType.DMA((_IB,)),
            pltpu.SemaphoreType.DMA((_RB,)),
            pltpu.SemaphoreType.DMA((_RB,)),
            pltpu.VMEM_SHARED((_NPAD, _H), jnp.float32),
        ],
    )(h, eidx, zeros_k)


def _pool_body(h_hbm, batch_hbm, ones_hbm, zeros_hbm, outp_hbm, outc_hbm,
               bidx_v, hrows_v, ones_v, zbuf_v, pool_sp, cnt_sp):
    c = lax.axis_index("c")
    s = lax.axis_index("s")
    w = c * _NS + s
    pltpu.sync_copy(batch_hbm.at[w], bidx_v)
    pltpu.sync_copy(ones_hbm, ones_v)
    pltpu.sync_copy(zeros_hbm, zbuf_v)
    pltpu.sync_copy(zbuf_v, pool_sp.at[pl.ds(s * _PSTRIPE, _PSTRIPE)])
    pltpu.sync_copy(zbuf_v, cnt_sp.at[pl.ds(s * _PSTRIPE, _PSTRIPE)])
    plsc.subcore_barrier()
    base = w * (_PC * _PK)
    for j in range(_PC):
        pltpu.sync_copy(h_hbm.at[pl.ds(base + j * _PK, _PK)], hrows_v)
        pltpu.sync_copy(hrows_v, pool_sp.at[bidx_v.at[j]], add=True)
        pltpu.sync_copy(ones_v, cnt_sp.at[bidx_v.at[j]], add=True)
    plsc.subcore_barrier()
    pltpu.sync_copy(pool_sp.at[pl.ds(s * _PSTRIPE, _PSTRIPE)], zbuf_v)
    pltpu.sync_copy(zbuf_v, outp_hbm.at[c, pl.ds(s * _PSTRIPE, _PSTRIPE)])
    pltpu.sync_copy(cnt_sp.at[pl.ds(s * _PSTRIPE, _PSTRIPE)], zbuf_v)
    pltpu.sync_copy(zbuf_v, outc_hbm.at[c, pl.ds(s * _PSTRIPE, _PSTRIPE)])


def _pool(h, bidx, ones_pk, zeros_ps):
    return pl.kernel(
        _pool_body,
        out_type=(
            jax.ShapeDtypeStruct((_NC, _PROWS, _H), jnp.float32),
            jax.ShapeDtypeStruct((_NC, _PROWS, _H), jnp.float32),
        ),
        mesh=_sc_mesh(),
        scratch_types=[
            pltpu.VMEM((_PC, _PK), jnp.int32),
            pltpu.VMEM((_PK, _H), jnp.float32),
            pltpu.VMEM((_PK, _H), jnp.float32),
            pltpu.VMEM((_PSTRIPE, _H), jnp.float32),
            pltpu.VMEM_SHARED((_PROWS, _H), jnp.float32),
            pltpu.VMEM_SHARED((_PROWS, _H), jnp.float32),
        ],
    )(h, bidx, ones_pk, zeros_ps)


_BLK = 1024


def _dense_in_body(x_ref, w_ref, b_ref, o_ref):
    o_ref[...] = (
        jnp.dot(x_ref[...], w_ref[...], preferred_element_type=jnp.float32)
        + b_ref[...]
    )


def _dense_in(x, w, b):
    return pl.pallas_call(
        _dense_in_body,
        grid=(_NPAD // _BLK,),
        in_specs=[
            pl.BlockSpec((_BLK, _H), lambda i: (i, 0)),
            pl.BlockSpec((_H, _H), lambda i: (0, 0)),
            pl.BlockSpec((1, _H), lambda i: (0, 0)),
        ],
        out_specs=pl.BlockSpec((_BLK, _H), lambda i: (i, 0)),
        out_shape=jax.ShapeDtypeStruct((_NPAD, _H), jnp.float32),
    )(x, w, b.reshape(1, _H))


def _conv_body(h_ref, a_ref, wr_ref, wn_ref, b_ref, g_ref, bb_ref, o_ref):
    h = h_ref[...]
    a = a_ref[0] + a_ref[1]
    y = (
        jnp.dot(h, wr_ref[...], preferred_element_type=jnp.float32)
        + jnp.dot(a, wn_ref[...], preferred_element_type=jnp.float32)
        + b_ref[...]
    )
    mu = jnp.mean(y, axis=-1, keepdims=True)
    d = y - mu
    var = jnp.mean(d * d, axis=-1, keepdims=True)
    y = d * lax.rsqrt(var + 1e-5) * g_ref[...] + bb_ref[...]
    o_ref[...] = jnp.maximum(y, 0.0)


def _conv(h, agg2, wr, wn, b, g, bb):
    return pl.pallas_call(
        _conv_body,
        grid=(_NPAD // _BLK,),
        in_specs=[
            pl.BlockSpec((_BLK, _H), lambda i: (i, 0)),
            pl.BlockSpec((_NC, _BLK, _H), lambda i: (0, i, 0)),
            pl.BlockSpec((_H, _H), lambda i: (0, 0)),
            pl.BlockSpec((_H, _H), lambda i: (0, 0)),
            pl.BlockSpec((1, _H), lambda i: (0, 0)),
            pl.BlockSpec((1, _H), lambda i: (0, 0)),
            pl.BlockSpec((1, _H), lambda i: (0, 0)),
        ],
        out_specs=pl.BlockSpec((_BLK, _H), lambda i: (i, 0)),
        out_shape=jax.ShapeDtypeStruct((_NPAD, _H), jnp.float32),
    )(h, agg2, wr, wn, b.reshape(1, _H), g.reshape(1, _H), bb.reshape(1, _H))


def _final_body(pp_ref, cc_ref, w_ref, b_ref, o_ref):
    p = pp_ref[0] + pp_ref[1]
    cnt = cc_ref[0] + cc_ref[1]
    pooled = p[:_G] / jnp.clip(cnt[:_G, 0:1], 1.0, None)
    o_ref[...] = (
        jnp.dot(pooled, w_ref[...], preferred_element_type=jnp.float32)
        + b_ref[...]
    )


def _final(pp, cc, w, b):
    return pl.pallas_call(
        _final_body,
        out_shape=jax.ShapeDtypeStruct((_G, _H), jnp.float32),
    )(pp, cc, w, b.reshape(1, _H))


def kernel(x, edge_index, batch, W_in, b_in, W_root, W_neigh, b_conv, ln_g, ln_b, W_out, b_out):
    f32 = jnp.float32
    x_pad = jnp.concatenate([x, jnp.zeros((_NPAD - _N, x.shape[1]), x.dtype)], axis=0)
    pad_ids = _N + (jnp.arange(_EPAD - _E, dtype=jnp.int32) % (_NPAD - _N))
    src_pad = jnp.concatenate([edge_index[0].astype(jnp.int32), pad_ids])
    dst_pad = jnp.concatenate([edge_index[1].astype(jnp.int32), pad_ids])
    eidx = jnp.stack(
        [src_pad.reshape(_NW, _C, _K), dst_pad.reshape(_NW, _C, _K)], axis=2
    )
    bidx = jnp.concatenate(
        [batch.astype(jnp.int32), jnp.full((_NPAD - _N,), _G, jnp.int32)]
    ).reshape(_NW, _PC, _PK)
    zeros_k = jnp.zeros((_K, _H), f32)
    zeros_ps = jnp.zeros((_PSTRIPE, _H), f32)
    ones_pk = jnp.ones((_PK, _H), f32)

    h = _dense_in(x_pad, W_in, b_in)
    for l in range(_L):
        agg2 = _edge_agg(h, eidx, zeros_k)
        h = _conv(h, agg2, W_root[l], W_neigh[l], b_conv[l], ln_g[l], ln_b[l])
    pp, cc = _pool(h, bidx, ones_pk, zeros_ps)
    return _final(pp, cc, W_out, b_out)


# speedup vs baseline: 11.6462x; 1.0354x over previous
import functools

import jax
import jax.numpy as jnp
from jax import lax
from jax.experimental import pallas as pl
from jax.experimental.pallas import tpu as pltpu
from jax.experimental.pallas import tpu_sc as plsc

_N = 10000
_E = 320000
_H = 128
_L = 3
_G = 256

_NPAD = 10240
_NC = 2
_NS = 16
_NW = _NC * _NS
_K = 64
_C = 160
_RB = 5
_GD = 4
_IB = 10
_EPAD = _NW * _C * _K

_PK = 80
_PC = 4
_PROWS = 384
_PSTRIPE = _PROWS // _NS

_STRIPE = _NPAD // _NS


def _sc_mesh():
    return plsc.VectorSubcoreMesh(
        core_axis_name="c", subcore_axis_name="s", num_cores=_NC, num_subcores=_NS
    )


def _edge_agg_body(h_hbm, eidx_hbm, zeros_hbm, out_hbm, idx_v, rows_v,
                   semi, semg, sems, agg_sp):
    c = lax.axis_index("c")
    s = lax.axis_index("s")
    w = c * _NS + s
    row0 = s * _STRIPE
    nz = _STRIPE // _K
    for t in range(nz):
        pltpu.async_copy(zeros_hbm, agg_sp.at[pl.ds(row0 + t * _K, _K)], sems.at[t % _RB])

    def idx_load(j, slot):
        pltpu.async_copy(eidx_hbm.at[w, j], idx_v.at[slot], semi.at[slot])

    def idx_wait(j, slot):
        pltpu.make_async_copy(eidx_hbm.at[w, j], idx_v.at[slot], semi.at[slot]).wait()

    def gather(j, u8, r):
        pltpu.async_copy(h_hbm.at[idx_v.at[u8, 0]], rows_v.at[r], semg.at[r])

    def gather_wait(u8, r):
        pltpu.make_async_copy(h_hbm.at[idx_v.at[u8, 0]], rows_v.at[r], semg.at[r]).wait()

    def scatter(u8, r):
        pltpu.async_copy(rows_v.at[r], agg_sp.at[idx_v.at[u8, 1]], sems.at[r], add=True)

    def scatter_wait(r):
        pltpu.make_async_copy(rows_v.at[r], agg_sp.at[idx_v.at[0, 1]], sems.at[r]).wait()

    def iteration(j, u, skip_b=False, skip_ef=False, do_idx=True):
        idx_wait(j, u)
        if not skip_b:
            scatter_wait(u % _RB)
        gather(j, u, u % _RB)
        if do_idx:
            idx_load(j + _RB, (u + _RB) % _IB)
        if not skip_ef:
            ug = (u - _GD) % _IB
            gather_wait(ug, ug % _RB)
            scatter(ug, ug % _RB)

    for t in range(_RB):
        idx_load(t, t)
    for t in range(_GD):
        iteration(t, t, skip_b=True, skip_ef=True)
    for t in range(nz):
        pltpu.make_async_copy(zeros_hbm, agg_sp.at[pl.ds(row0 + t * _K, _K)], sems.at[t % _RB]).wait()
    plsc.subcore_barrier()
    for t in range(_GD, _IB):
        iteration(t, t, skip_b=(t < _RB))

    def body(bb, carry):
        j0 = _IB + _IB * bb
        for u in range(_IB):
            iteration(j0 + u, u)
        return carry

    lax.fori_loop(0, (_C - 2 * _IB) // _IB, body, 0)
    for t in range(_IB):
        j = _C - _IB + t
        iteration(j, t, do_idx=(t < _IB - _RB))
    for t in range(_GD):
        j = _C - _GD + t
        u = j % _IB
        gather_wait(u, u % _RB)
        scatter(u, u % _RB)
    for r in range(_RB):
        scatter_wait(r)
    plsc.subcore_barrier()
    pltpu.sync_copy(agg_sp.at[pl.ds(row0, _STRIPE)], out_hbm.at[c, pl.ds(row0, _STRIPE)])


def _edge_agg(h, eidx, zeros_k):
    return pl.kernel(
        _edge_agg_body,
        out_type=jax.ShapeDtypeStruct((_NC, _NPAD, _H), jnp.float32),
        mesh=_sc_mesh(),
        scratch_types=[
            pltpu.VMEM((_IB, 2, _K), jnp.int32),
            pltpu.VMEM((_RB, _K, _H), jnp.float32),
            pltpu.SemaphoreType.DMA((_IB,)),
            pltpu.SemaphoreType.DMA((_RB,)),
            pltpu.SemaphoreType.DMA((_RB,)),
            pltpu.VMEM_SHARED((_NPAD, _H), jnp.float32),
        ],
    )(h, eidx, zeros_k)


def _pool_body(h_hbm, batch_hbm, ones_hbm, zeros_hbm, outp_hbm, outc_hbm,
               bidx_v, hrows_v, ones_v, zbuf_v, pool_sp, cnt_sp):
    c = lax.axis_index("c")
    s = lax.axis_index("s")
    w = c * _NS + s
    pltpu.sync_copy(batch_hbm.at[w], bidx_v)
    pltpu.sync_copy(ones_hbm, ones_v)
    pltpu.sync_copy(zeros_hbm, zbuf_v)
    pltpu.sync_copy(zbuf_v, pool_sp.at[pl.ds(s * _PSTRIPE, _PSTRIPE)])
    pltpu.sync_copy(zbuf_v, cnt_sp.at[pl.ds(s * _PSTRIPE, _PSTRIPE)])
    plsc.subcore_barrier()
    base = w * (_PC * _PK)
    for j in range(_PC):
        pltpu.sync_copy(h_hbm.at[pl.ds(base + j * _PK, _PK)], hrows_v)
        pltpu.sync_copy(hrows_v, pool_sp.at[bidx_v.at[j]], add=True)
        pltpu.sync_copy(ones_v, cnt_sp.at[bidx_v.at[j]], add=True)
    plsc.subcore_barrier()
    pltpu.sync_copy(pool_sp.at[pl.ds(s * _PSTRIPE, _PSTRIPE)], zbuf_v)
    pltpu.sync_copy(zbuf_v, outp_hbm.at[c, pl.ds(s * _PSTRIPE, _PSTRIPE)])
    pltpu.sync_copy(cnt_sp.at[pl.ds(s * _PSTRIPE, _PSTRIPE)], zbuf_v)
    pltpu.sync_copy(zbuf_v, outc_hbm.at[c, pl.ds(s * _PSTRIPE, _PSTRIPE)])


def _pool(h, bidx, ones_pk, zeros_ps):
    return pl.kernel(
        _pool_body,
        out_type=(
            jax.ShapeDtypeStruct((_NC, _PROWS, _H), jnp.float32),
            jax.ShapeDtypeStruct((_NC, _PROWS, _H), jnp.float32),
        ),
        mesh=_sc_mesh(),
        scratch_types=[
            pltpu.VMEM((_PC, _PK), jnp.int32),
            pltpu.VMEM((_PK, _H), jnp.float32),
            pltpu.VMEM((_PK, _H), jnp.float32),
            pltpu.VMEM((_PSTRIPE, _H), jnp.float32),
            pltpu.VMEM_SHARED((_PROWS, _H), jnp.float32),
            pltpu.VMEM_SHARED((_PROWS, _H), jnp.float32),
        ],
    )(h, bidx, ones_pk, zeros_ps)


_BLK = 1024


def _dense_in_body(x_ref, w_ref, b_ref, o_ref):
    o_ref[...] = (
        jnp.dot(x_ref[...], w_ref[...], preferred_element_type=jnp.float32)
        + b_ref[...]
    )


def _dense_in(x, w, b):
    return pl.pallas_call(
        _dense_in_body,
        grid=(_NPAD // _BLK,),
        in_specs=[
            pl.BlockSpec((_BLK, _H), lambda i: (i, 0)),
            pl.BlockSpec((_H, _H), lambda i: (0, 0)),
            pl.BlockSpec((1, _H), lambda i: (0, 0)),
        ],
        out_specs=pl.BlockSpec((_BLK, _H), lambda i: (i, 0)),
        out_shape=jax.ShapeDtypeStruct((_NPAD, _H), jnp.float32),
    )(x, w, b.reshape(1, _H))


def _conv_body(h_ref, a_ref, wr_ref, wn_ref, b_ref, g_ref, bb_ref, o_ref):
    h = h_ref[...]
    a = a_ref[0] + a_ref[1]
    y = (
        jnp.dot(h, wr_ref[...], preferred_element_type=jnp.float32)
        + jnp.dot(a, wn_ref[...], preferred_element_type=jnp.float32)
        + b_ref[...]
    )
    mu = jnp.mean(y, axis=-1, keepdims=True)
    d = y - mu
    var = jnp.mean(d * d, axis=-1, keepdims=True)
    y = d * lax.rsqrt(var + 1e-5) * g_ref[...] + bb_ref[...]
    o_ref[...] = jnp.maximum(y, 0.0)


def _conv(h, agg2, wr, wn, b, g, bb):
    return pl.pallas_call(
        _conv_body,
        grid=(_NPAD // _BLK,),
        in_specs=[
            pl.BlockSpec((_BLK, _H), lambda i: (i, 0)),
            pl.BlockSpec((_NC, _BLK, _H), lambda i: (0, i, 0)),
            pl.BlockSpec((_H, _H), lambda i: (0, 0)),
            pl.BlockSpec((_H, _H), lambda i: (0, 0)),
            pl.BlockSpec((1, _H), lambda i: (0, 0)),
            pl.BlockSpec((1, _H), lambda i: (0, 0)),
            pl.BlockSpec((1, _H), lambda i: (0, 0)),
        ],
        out_specs=pl.BlockSpec((_BLK, _H), lambda i: (i, 0)),
        out_shape=jax.ShapeDtypeStruct((_NPAD, _H), jnp.float32),
    )(h, agg2, wr, wn, b.reshape(1, _H), g.reshape(1, _H), bb.reshape(1, _H))


def _final_body(pp_ref, cc_ref, w_ref, b_ref, o_ref):
    p = pp_ref[0] + pp_ref[1]
    cnt = cc_ref[0] + cc_ref[1]
    pooled = p[:_G] / jnp.clip(cnt[:_G, 0:1], 1.0, None)
    o_ref[...] = (
        jnp.dot(pooled, w_ref[...], preferred_element_type=jnp.float32)
        + b_ref[...]
    )


def _final(pp, cc, w, b):
    return pl.pallas_call(
        _final_body,
        out_shape=jax.ShapeDtypeStruct((_G, _H), jnp.float32),
    )(pp, cc, w, b.reshape(1, _H))


def kernel(x, edge_index, batch, W_in, b_in, W_root, W_neigh, b_conv, ln_g, ln_b, W_out, b_out):
    f32 = jnp.float32
    x_pad = jnp.concatenate([x, jnp.zeros((_NPAD - _N, x.shape[1]), x.dtype)], axis=0)
    pad_ids = _N + (jnp.arange(_EPAD - _E, dtype=jnp.int32) % (_NPAD - _N))
    src_pad = jnp.concatenate([edge_index[0].astype(jnp.int32), pad_ids])
    dst_pad = jnp.concatenate([edge_index[1].astype(jnp.int32), pad_ids])
    eidx = jnp.stack(
        [src_pad.reshape(_NW, _C, _K), dst_pad.reshape(_NW, _C, _K)], axis=2
    )
    bidx = jnp.concatenate(
        [batch.astype(jnp.int32), jnp.full((_NPAD - _N,), _G, jnp.int32)]
    ).reshape(_NW, _PC, _PK)
    zeros_k = jnp.zeros((_K, _H), f32)
    zeros_ps = jnp.zeros((_PSTRIPE, _H), f32)
    ones_pk = jnp.ones((_PK, _H), f32)

    h = _dense_in(x_pad, W_in, b_in)
    for l in range(_L):
        agg2 = _edge_agg(h, eidx, zeros_k)
        h = _conv(h, agg2, W_root[l], W_neigh[l], b_conv[l], ln_g[l], ln_b[l])
    pp, cc = _pool(h, bidx, ones_pk, zeros_ps)
    return _final(pp, cc, W_out, b_out)


# speedup vs baseline: 12.0775x; 1.0370x over previous
import functools

import jax
import jax.numpy as jnp
from jax import lax
from jax.experimental import pallas as pl
from jax.experimental.pallas import tpu as pltpu
from jax.experimental.pallas import tpu_sc as plsc

_N = 10000
_E = 320000
_H = 128
_L = 3
_G = 256

_NPAD = 10240
_NC = 2
_NS = 16
_NW = _NC * _NS
_K = 80
_C = 128
_RB = 4
_GD = 3
_IB = 8
_EPAD = _NW * _C * _K

_PK = 80
_PC = 4
_PROWS = 384
_PSTRIPE = _PROWS // _NS

_STRIPE = _NPAD // _NS


def _sc_mesh():
    return plsc.VectorSubcoreMesh(
        core_axis_name="c", subcore_axis_name="s", num_cores=_NC, num_subcores=_NS
    )


def _edge_agg_body(h_hbm, eidx_hbm, zeros_hbm, out_hbm, idx_v, rows_v,
                   semi, semg, sems, agg_sp):
    c = lax.axis_index("c")
    s = lax.axis_index("s")
    w = c * _NS + s
    row0 = s * _STRIPE
    nz = _STRIPE // _K
    for t in range(nz):
        pltpu.async_copy(zeros_hbm, agg_sp.at[pl.ds(row0 + t * _K, _K)], sems.at[t % _RB])

    def idx_load(j, slot):
        pltpu.async_copy(eidx_hbm.at[w, j], idx_v.at[slot], semi.at[slot])

    def idx_wait(j, slot):
        pltpu.make_async_copy(eidx_hbm.at[w, j], idx_v.at[slot], semi.at[slot]).wait()

    def gather(j, u8, r):
        pltpu.async_copy(h_hbm.at[idx_v.at[u8, 0]], rows_v.at[r], semg.at[r])

    def gather_wait(u8, r):
        pltpu.make_async_copy(h_hbm.at[idx_v.at[u8, 0]], rows_v.at[r], semg.at[r]).wait()

    def scatter(u8, r):
        pltpu.async_copy(rows_v.at[r], agg_sp.at[idx_v.at[u8, 1]], sems.at[r], add=True)

    def scatter_wait(r):
        pltpu.make_async_copy(rows_v.at[r], agg_sp.at[idx_v.at[0, 1]], sems.at[r]).wait()

    def iteration(j, u, skip_b=False, skip_ef=False, do_idx=True):
        idx_wait(j, u)
        if not skip_b:
            scatter_wait(u % _RB)
        gather(j, u, u % _RB)
        if do_idx:
            idx_load(j + _RB, (u + _RB) % _IB)
        if not skip_ef:
            ug = (u - _GD) % _IB
            gather_wait(ug, ug % _RB)
            scatter(ug, ug % _RB)

    for t in range(_RB):
        idx_load(t, t)
    for t in range(_GD):
        iteration(t, t, skip_b=True, skip_ef=True)
    for t in range(nz):
        pltpu.make_async_copy(zeros_hbm, agg_sp.at[pl.ds(row0 + t * _K, _K)], sems.at[t % _RB]).wait()
    plsc.subcore_barrier()
    for t in range(_GD, _IB):
        iteration(t, t, skip_b=(t < _RB))

    def body(bb, carry):
        j0 = _IB + _IB * bb
        for u in range(_IB):
            iteration(j0 + u, u)
        return carry

    lax.fori_loop(0, (_C - 2 * _IB) // _IB, body, 0)
    for t in range(_IB):
        j = _C - _IB + t
        iteration(j, t, do_idx=(t < _IB - _RB))
    for t in range(_GD):
        j = _C - _GD + t
        u = j % _IB
        gather_wait(u, u % _RB)
        scatter(u, u % _RB)
    for r in range(_RB):
        scatter_wait(r)
    plsc.subcore_barrier()
    pltpu.sync_copy(agg_sp.at[pl.ds(row0, _STRIPE)], out_hbm.at[c, pl.ds(row0, _STRIPE)])


def _edge_agg(h, eidx, zeros_k):
    return pl.kernel(
        _edge_agg_body,
        out_type=jax.ShapeDtypeStruct((_NC, _NPAD, _H), jnp.float32),
        mesh=_sc_mesh(),
        scratch_types=[
            pltpu.VMEM((_IB, 2, _K), jnp.int32),
            pltpu.VMEM((_RB, _K, _H), jnp.float32),
            pltpu.SemaphoreType.DMA((_IB,)),
            pltpu.SemaphoreType.DMA((_RB,)),
            pltpu.SemaphoreType.DMA((_RB,)),
            pltpu.VMEM_SHARED((_NPAD, _H), jnp.float32),
        ],
    )(h, eidx, zeros_k)


def _pool_body(h_hbm, batch_hbm, ones_hbm, zeros_hbm, outp_hbm, outc_hbm,
               bidx_v, hrows_v, ones_v, zbuf_v, pool_sp, cnt_sp):
    c = lax.axis_index("c")
    s = lax.axis_index("s")
    w = c * _NS + s
    pltpu.sync_copy(batch_hbm.at[w], bidx_v)
    pltpu.sync_copy(ones_hbm, ones_v)
    pltpu.sync_copy(zeros_hbm, zbuf_v)
    pltpu.sync_copy(zbuf_v, pool_sp.at[pl.ds(s * _PSTRIPE, _PSTRIPE)])
    pltpu.sync_copy(zbuf_v, cnt_sp.at[pl.ds(s * _PSTRIPE, _PSTRIPE)])
    plsc.subcore_barrier()
    base = w * (_PC * _PK)
    for j in range(_PC):
        pltpu.sync_copy(h_hbm.at[pl.ds(base + j * _PK, _PK)], hrows_v)
        pltpu.sync_copy(hrows_v, pool_sp.at[bidx_v.at[j]], add=True)
        pltpu.sync_copy(ones_v, cnt_sp.at[bidx_v.at[j]], add=True)
    plsc.subcore_barrier()
    pltpu.sync_copy(pool_sp.at[pl.ds(s * _PSTRIPE, _PSTRIPE)], zbuf_v)
    pltpu.sync_copy(zbuf_v, outp_hbm.at[c, pl.ds(s * _PSTRIPE, _PSTRIPE)])
    pltpu.sync_copy(cnt_sp.at[pl.ds(s * _PSTRIPE, _PSTRIPE)], zbuf_v)
    pltpu.sync_copy(zbuf_v, outc_hbm.at[c, pl.ds(s * _PSTRIPE, _PSTRIPE)])


def _pool(h, bidx, ones_pk, zeros_ps):
    return pl.kernel(
        _pool_body,
        out_type=(
            jax.ShapeDtypeStruct((_NC, _PROWS, _H), jnp.float32),
            jax.ShapeDtypeStruct((_NC, _PROWS, _H), jnp.float32),
        ),
        mesh=_sc_mesh(),
        scratch_types=[
            pltpu.VMEM((_PC, _PK), jnp.int32),
            pltpu.VMEM((_PK, _H), jnp.float32),
            pltpu.VMEM((_PK, _H), jnp.float32),
            pltpu.VMEM((_PSTRIPE, _H), jnp.float32),
            pltpu.VMEM_SHARED((_PROWS, _H), jnp.float32),
            pltpu.VMEM_SHARED((_PROWS, _H), jnp.float32),
        ],
    )(h, bidx, ones_pk, zeros_ps)


_BLK = 1024


def _dense_in_body(x_ref, w_ref, b_ref, o_ref):
    o_ref[...] = (
        jnp.dot(x_ref[...], w_ref[...], preferred_element_type=jnp.float32)
        + b_ref[...]
    )


def _dense_in(x, w, b):
    return pl.pallas_call(
        _dense_in_body,
        grid=(_NPAD // _BLK,),
        in_specs=[
            pl.BlockSpec((_BLK, _H), lambda i: (i, 0)),
            pl.BlockSpec((_H, _H), lambda i: (0, 0)),
            pl.BlockSpec((1, _H), lambda i: (0, 0)),
        ],
        out_specs=pl.BlockSpec((_BLK, _H), lambda i: (i, 0)),
        out_shape=jax.ShapeDtypeStruct((_NPAD, _H), jnp.float32),
    )(x, w, b.reshape(1, _H))


def _conv_body(h_ref, a_ref, wr_ref, wn_ref, b_ref, g_ref, bb_ref, o_ref):
    h = h_ref[...]
    a = a_ref[0] + a_ref[1]
    y = (
        jnp.dot(h, wr_ref[...], preferred_element_type=jnp.float32)
        + jnp.dot(a, wn_ref[...], preferred_element_type=jnp.float32)
        + b_ref[...]
    )
    mu = jnp.mean(y, axis=-1, keepdims=True)
    d = y - mu
    var = jnp.mean(d * d, axis=-1, keepdims=True)
    y = d * lax.rsqrt(var + 1e-5) * g_ref[...] + bb_ref[...]
    o_ref[...] = jnp.maximum(y, 0.0)


def _conv(h, agg2, wr, wn, b, g, bb):
    return pl.pallas_call(
        _conv_body,
        grid=(_NPAD // _BLK,),
        in_specs=[
            pl.BlockSpec((_BLK, _H), lambda i: (i, 0)),
            pl.BlockSpec((_NC, _BLK, _H), lambda i: (0, i, 0)),
            pl.BlockSpec((_H, _H), lambda i: (0, 0)),
            pl.BlockSpec((_H, _H), lambda i: (0, 0)),
            pl.BlockSpec((1, _H), lambda i: (0, 0)),
            pl.BlockSpec((1, _H), lambda i: (0, 0)),
            pl.BlockSpec((1, _H), lambda i: (0, 0)),
        ],
        out_specs=pl.BlockSpec((_BLK, _H), lambda i: (i, 0)),
        out_shape=jax.ShapeDtypeStruct((_NPAD, _H), jnp.float32),
    )(h, agg2, wr, wn, b.reshape(1, _H), g.reshape(1, _H), bb.reshape(1, _H))


def _final_body(pp_ref, cc_ref, w_ref, b_ref, o_ref):
    p = pp_ref[0] + pp_ref[1]
    cnt = cc_ref[0] + cc_ref[1]
    pooled = p[:_G] / jnp.clip(cnt[:_G, 0:1], 1.0, None)
    o_ref[...] = (
        jnp.dot(pooled, w_ref[...], preferred_element_type=jnp.float32)
        + b_ref[...]
    )


def _final(pp, cc, w, b):
    return pl.pallas_call(
        _final_body,
        out_shape=jax.ShapeDtypeStruct((_G, _H), jnp.float32),
    )(pp, cc, w, b.reshape(1, _H))


def kernel(x, edge_index, batch, W_in, b_in, W_root, W_neigh, b_conv, ln_g, ln_b, W_out, b_out):
    f32 = jnp.float32
    x_pad = jnp.concatenate([x, jnp.zeros((_NPAD - _N, x.shape[1]), x.dtype)], axis=0)
    pad_ids = _N + (jnp.arange(_EPAD - _E, dtype=jnp.int32) % (_NPAD - _N))
    src_pad = jnp.concatenate([edge_index[0].astype(jnp.int32), pad_ids])
    dst_pad = jnp.concatenate([edge_index[1].astype(jnp.int32), pad_ids])
    eidx = jnp.stack(
        [src_pad.reshape(_NW, _C, _K), dst_pad.reshape(_NW, _C, _K)], axis=2
    )
    bidx = jnp.concatenate(
        [batch.astype(jnp.int32), jnp.full((_NPAD - _N,), _G, jnp.int32)]
    ).reshape(_NW, _PC, _PK)
    zeros_k = jnp.zeros((_K, _H), f32)
    zeros_ps = jnp.zeros((_PSTRIPE, _H), f32)
    ones_pk = jnp.ones((_PK, _H), f32)

    h = _dense_in(x_pad, W_in, b_in)
    for l in range(_L):
        agg2 = _edge_agg(h, eidx, zeros_k)
        h = _conv(h, agg2, W_root[l], W_neigh[l], b_conv[l], ln_g[l], ln_b[l])
    pp, cc = _pool(h, bidx, ones_pk, zeros_ps)
    return _final(pp, cc, W_out, b_out)
